# baseline (speedup 1.0000x reference)
import jax
import jax.numpy as jnp
from jax.experimental import pallas as pl
from jax.experimental.pallas import tpu as pltpu

_BI = 32


def _row_block_kernel(x_ref, rev_ref, o_ref):
    i0 = pl.program_id(0) * _BI
    seq = x_ref.shape[0]
    for di in range(_BI):
        start = (seq - 1) - (i0 + di)
        o_ref[di] = x_ref[...] + rev_ref[pl.ds(start, seq), :]


def kernel(x, table):
    seq = x.shape[1]
    d = x.shape[2]
    maxlen = (table.shape[0] + 1) // 2
    assert maxlen == seq
    rev = jnp.flip(table, axis=0)
    rev = jnp.pad(rev, ((0, 1), (0, 0)))
    x2 = x[0]

    out = pl.pallas_call(
        _row_block_kernel,
        grid=(seq // _BI,),
        in_specs=[
            pl.BlockSpec((seq, d), lambda i: (0, 0)),
            pl.BlockSpec((2 * seq, d), lambda i: (0, 0)),
        ],
        out_specs=pl.BlockSpec((_BI, seq, d), lambda i: (i, 0, 0)),
        out_shape=jax.ShapeDtypeStruct((seq, seq, d), x.dtype),
    )(x2, rev)
    return out


# speedup vs baseline: 1.0015x; 1.0015x over previous
import jax
import jax.numpy as jnp
from jax.experimental import pallas as pl
from jax.experimental.pallas import tpu as pltpu

_BI = 16


def _row_block_kernel(x_ref, rev_ref, o_ref):
    i0 = pl.program_id(0) * _BI
    seq = x_ref.shape[0]
    for di in range(_BI):
        start = (seq - 1) - (i0 + di)
        o_ref[di] = x_ref[...] + rev_ref[pl.ds(start, seq), :]


def kernel(x, table):
    seq = x.shape[1]
    d = x.shape[2]
    maxlen = (table.shape[0] + 1) // 2
    assert maxlen == seq
    rev = jnp.flip(table, axis=0)
    rev = jnp.pad(rev, ((0, 1), (0, 0)))
    x2 = x[0]

    out = pl.pallas_call(
        _row_block_kernel,
        grid=(seq // _BI,),
        in_specs=[
            pl.BlockSpec((seq, d), lambda i: (0, 0)),
            pl.BlockSpec((2 * seq, d), lambda i: (0, 0)),
        ],
        out_specs=pl.BlockSpec((_BI, seq, d), lambda i: (i, 0, 0)),
        out_shape=jax.ShapeDtypeStruct((seq, seq, d), x.dtype),
    )(x2, rev)
    return out
